# baseline (speedup 1.0000x reference)
import functools

import jax
import jax.numpy as jnp
from jax import lax
from jax.experimental import pallas as pl
from jax.experimental.pallas import tpu as pltpu
from jax.experimental.pallas import tpu_sc as plsc

N = 8192
HID = 64
K_NN = 36
N_CLS = 17
TBL = 256

PROJ_B = 512
TOPK_B = 512
MLP_B = 128

NUM_WORKERS = 32
GATHER_CHUNK = 128


def _proj_body(feat_ref, wq_ref, wk_ref, wv_ref, wp1_ref, q_ref, pq_ref,
               tbl_ref):
  f = feat_ref[...]
  q_ref[...] = jnp.dot(f, wq_ref[...], preferred_element_type=jnp.float32)
  kf = jnp.dot(f, wk_ref[...], preferred_element_type=jnp.float32)
  vf = jnp.dot(f, wv_ref[...], preferred_element_type=jnp.float32)
  p = jnp.dot(f[:, 0:3], wp1_ref[...], preferred_element_type=jnp.float32)
  pq_ref[...] = p
  tbl_ref[:, 0:HID] = kf
  tbl_ref[:, HID:2 * HID] = vf
  tbl_ref[:, 2 * HID:3 * HID] = p
  tbl_ref[:, 3 * HID:TBL] = jnp.zeros((f.shape[0], HID), jnp.float32)


def _proj(feat, w_q, w_k, w_v, w_pos1):
  grid = (N // PROJ_B,)
  return pl.pallas_call(
      _proj_body,
      grid=grid,
      in_specs=[
          pl.BlockSpec((PROJ_B, 6), lambda b: (b, 0)),
          pl.BlockSpec((6, HID), lambda b: (0, 0)),
          pl.BlockSpec((6, HID), lambda b: (0, 0)),
          pl.BlockSpec((6, HID), lambda b: (0, 0)),
          pl.BlockSpec((3, HID), lambda b: (0, 0)),
      ],
      out_specs=[
          pl.BlockSpec((PROJ_B, HID), lambda b: (b, 0)),
          pl.BlockSpec((PROJ_B, HID), lambda b: (b, 0)),
          pl.BlockSpec((PROJ_B, TBL), lambda b: (b, 0)),
      ],
      out_shape=[
          jax.ShapeDtypeStruct((N, HID), jnp.float32),
          jax.ShapeDtypeStruct((N, HID), jnp.float32),
          jax.ShapeDtypeStruct((N, TBL), jnp.float32),
      ],
  )(feat, w_q, w_k, w_v, w_pos1)


POOL_T = 6


def _flat_topk(d, qn):
  iota = lax.broadcasted_iota(jnp.int32, (qn, N), 1)
  kiota = lax.broadcasted_iota(jnp.int32, (qn, K_NN), 1)
  idxbuf = jnp.zeros((qn, K_NN), jnp.int32)

  def body(r, carry):
    d, idxbuf = carry
    m = jnp.min(d, axis=1, keepdims=True)
    j = jnp.min(jnp.where(d <= m, iota, N), axis=1, keepdims=True)
    idxbuf = jnp.where(kiota == r, j, idxbuf)
    d = jnp.where(iota == j, jnp.inf, d)
    return d, idxbuf

  _, idxbuf = lax.fori_loop(0, K_NN, body, (d, idxbuf))
  return idxbuf


def _topk_body(xq_ref, xyzt_ref, idx_ref):
  qn = TOPK_B
  ns = N // 128
  ng = 128
  xq = xq_ref[...]
  xt = xyzt_ref[...]
  sqc = jnp.sum(xt * xt, axis=0, keepdims=True)
  d = sqc - 2.0 * jnp.dot(xq, xt, preferred_element_type=jnp.float32)
  d3 = d.reshape(qn, ns, ng)
  siota3 = lax.broadcasted_iota(jnp.int32, (qn, ns, ng), 1).astype(jnp.float32)
  kiota = lax.broadcasted_iota(jnp.int32, (qn, K_NN), 1)
  giota = lax.broadcasted_iota(jnp.int32, (qn, ng), 1).astype(jnp.float32)

  masked = d3
  ms, jids = [], []
  for _ in range(POOL_T):
    mt = jnp.min(masked, axis=1)
    hit = masked == mt[:, None, :]
    at = jnp.min(jnp.where(hit, siota3, float(ns)), axis=1)
    masked = jnp.where(hit & (siota3 == at[:, None, :]), jnp.inf, masked)
    ms.append(mt)
    jids.append(at * ng + giota)
  pool = jnp.concatenate(ms, axis=1)
  jpool = jnp.concatenate(jids, axis=1)
  pw = ng * POOL_T
  big = jnp.float32(N)
  idxbuf = jnp.zeros((qn, K_NN), jnp.float32)

  def body(r, carry):
    pool, idxbuf = carry
    m = jnp.min(pool, axis=1, keepdims=True)
    hit = pool <= m
    j = jnp.min(jnp.where(hit, jpool, big), axis=1, keepdims=True)
    idxbuf = jnp.where(kiota == r, j, idxbuf)
    pool = jnp.where(hit & (jpool == j), jnp.inf, pool)
    return pool, idxbuf

  pool, idxbuf = lax.fori_loop(0, K_NN, body, (pool, idxbuf))
  idx_ref[...] = idxbuf.astype(jnp.int32)

  @pl.when(jnp.any(pool[:, (POOL_T - 1) * ng:] == jnp.inf))
  def _():
    idx_ref[...] = _flat_topk(d, qn)


def _topk(xyz, xyzt, nq):
  grid = (nq // TOPK_B,)
  return pl.pallas_call(
      _topk_body,
      grid=grid,
      in_specs=[
          pl.BlockSpec((TOPK_B, 3), lambda b: (b, 0)),
          pl.BlockSpec((3, N), lambda b: (0, 0)),
      ],
      out_specs=pl.BlockSpec((TOPK_B, K_NN), lambda b: (b, 0)),
      out_shape=jax.ShapeDtypeStruct((nq, K_NN), jnp.int32),
  )(xyz, xyzt)


def _gather(tbl, flat_idx):
  b_total = flat_idx.shape[0]
  b_per_w = b_total // NUM_WORKERS
  n_chunks = b_per_w // GATHER_CHUNK
  mesh = plsc.VectorSubcoreMesh(core_axis_name="c", subcore_axis_name="s")

  @functools.partial(
      pl.kernel,
      out_type=jax.ShapeDtypeStruct((b_total, TBL), jnp.float32),
      mesh=mesh,
      scratch_types=[
          pltpu.VMEM((GATHER_CHUNK,), jnp.int32),
          pltpu.VMEM((GATHER_CHUNK, TBL), jnp.float32),
          pltpu.SemaphoreType.DMA,
      ],
  )
  def gk(tbl_hbm, idx_hbm, out_hbm, idx_v, rows_v, sem):
    wid = lax.axis_index("s") * 2 + lax.axis_index("c")
    base = wid * b_per_w

    def body(i, carry):
      off = base + i * GATHER_CHUNK
      pltpu.sync_copy(idx_hbm.at[pl.ds(off, GATHER_CHUNK)], idx_v)
      pltpu.async_copy(tbl_hbm.at[idx_v], rows_v, sem).wait()
      pltpu.sync_copy(rows_v, out_hbm.at[pl.ds(off, GATHER_CHUNK)])
      return carry

    lax.fori_loop(0, n_chunks, body, 0)

  return gk(tbl, flat_idx)


def _mlp_body(g_ref, q_ref, pq_ref, bp1_ref, wp2_ref, bp2_ref,
              wa1_ref, ba1_ref, wa2_ref, ba2_ref, wc_ref, bc_ref, sem_ref):
  g = g_ref[...]
  nb = MLP_B
  kf = g[:, :, 0:HID].reshape(K_NN * nb, HID)
  vf = g[:, :, HID:2 * HID].reshape(K_NN * nb, HID)
  pj = g[:, :, 2 * HID:3 * HID]
  pq = pq_ref[...]
  h = pj - pq[None, :, :] + bp1_ref[...].reshape(1, 1, HID)
  ph = jnp.maximum(h, 0.0).reshape(K_NN * nb, HID)
  pos = (jnp.dot(ph, wp2_ref[...], preferred_element_type=jnp.float32)
         + bp2_ref[...])
  q = q_ref[...]
  qrep = jnp.broadcast_to(q[None, :, :], (K_NN, nb, HID)).reshape(
      K_NN * nb, HID)
  a0 = qrep - kf + pos
  ah = jnp.maximum(
      jnp.dot(a0, wa1_ref[...], preferred_element_type=jnp.float32)
      + ba1_ref[...], 0.0)
  al = (jnp.dot(ah, wa2_ref[...], preferred_element_type=jnp.float32)
        + ba2_ref[...]).reshape(K_NN, nb, HID)
  m = jnp.max(al, axis=0, keepdims=True)
  e = jnp.exp(al - m)
  s = jnp.sum(e, axis=0, keepdims=True)
  attn = e / s
  vp = vf + pos
  out = jnp.sum(attn * vp.reshape(K_NN, nb, HID), axis=0)
  sem_ref[...] = (jnp.dot(out, wc_ref[...], preferred_element_type=jnp.float32)
                  + bc_ref[...])


def _mlp(g3, q, pq, bp1, wp2, bp2, wa1, ba1, wa2, ba2, wc, bc):
  nq = q.shape[0]
  grid = (nq // MLP_B,)
  full = lambda r, c: pl.BlockSpec((r, c), lambda b: (0, 0))
  return pl.pallas_call(
      _mlp_body,
      grid=grid,
      in_specs=[
          pl.BlockSpec((K_NN, MLP_B, TBL), lambda b: (0, b, 0)),
          pl.BlockSpec((MLP_B, HID), lambda b: (b, 0)),
          pl.BlockSpec((MLP_B, HID), lambda b: (b, 0)),
          full(1, HID), full(HID, HID), full(1, HID),
          full(HID, HID), full(1, HID), full(HID, HID), full(1, HID),
          full(HID, N_CLS), full(1, N_CLS),
      ],
      out_specs=pl.BlockSpec((MLP_B, N_CLS), lambda b: (b, 0)),
      out_shape=jax.ShapeDtypeStruct((nq, N_CLS), jnp.float32),
  )(g3, q, pq, bp1, wp2, bp2, wa1, ba1, wa2, ba2, wc, bc)


def kernel(feat, w_q, w_k, w_v, w_pos1, b_pos1, w_pos2, b_pos2,
           w_attn1, b_attn1, w_attn2, b_attn2, w_cls, b_cls):
  xyz = feat[:, :3]
  xyzt = xyz.T
  q, pq, tbl = _proj(feat, w_q, w_k, w_v, w_pos1)
  r1 = lambda b: b.reshape(1, -1)
  nh = N // 4
  outs = []
  for h in range(4):
    rows = slice(h * nh, (h + 1) * nh)
    idx = _topk(xyz[rows], xyzt, nh)
    flat_idx = idx.T.reshape(-1)
    gath = _gather(tbl, flat_idx)
    g3 = gath.reshape(K_NN, nh, TBL)
    outs.append(_mlp(g3, q[rows], pq[rows], r1(b_pos1), w_pos2, r1(b_pos2),
                     w_attn1, r1(b_attn1), w_attn2, r1(b_attn2),
                     w_cls, r1(b_cls)))
  return jnp.concatenate(outs, axis=0)


# speedup vs baseline: 1.1284x; 1.1284x over previous
import functools

import jax
import jax.numpy as jnp
from jax import lax
from jax.experimental import pallas as pl
from jax.experimental.pallas import tpu as pltpu
from jax.experimental.pallas import tpu_sc as plsc

N = 8192
HID = 64
K_NN = 36
N_CLS = 17
TBL = 256

PROJ_B = 512
TOPK_B = 256
MLP_B = 128

NUM_WORKERS = 32
GATHER_CHUNK = 128


def _proj_body(feat_ref, wq_ref, wk_ref, wv_ref, wp1_ref, q_ref, pq_ref,
               tbl_ref):
  f = feat_ref[...]
  q_ref[...] = jnp.dot(f, wq_ref[...], preferred_element_type=jnp.float32)
  kf = jnp.dot(f, wk_ref[...], preferred_element_type=jnp.float32)
  vf = jnp.dot(f, wv_ref[...], preferred_element_type=jnp.float32)
  p = jnp.dot(f[:, 0:3], wp1_ref[...], preferred_element_type=jnp.float32)
  pq_ref[...] = p
  tbl_ref[:, 0:HID] = kf
  tbl_ref[:, HID:2 * HID] = vf
  tbl_ref[:, 2 * HID:3 * HID] = p
  tbl_ref[:, 3 * HID:TBL] = jnp.zeros((f.shape[0], HID), jnp.float32)


def _proj(feat, w_q, w_k, w_v, w_pos1):
  grid = (N // PROJ_B,)
  return pl.pallas_call(
      _proj_body,
      grid=grid,
      in_specs=[
          pl.BlockSpec((PROJ_B, 6), lambda b: (b, 0)),
          pl.BlockSpec((6, HID), lambda b: (0, 0)),
          pl.BlockSpec((6, HID), lambda b: (0, 0)),
          pl.BlockSpec((6, HID), lambda b: (0, 0)),
          pl.BlockSpec((3, HID), lambda b: (0, 0)),
      ],
      out_specs=[
          pl.BlockSpec((PROJ_B, HID), lambda b: (b, 0)),
          pl.BlockSpec((PROJ_B, HID), lambda b: (b, 0)),
          pl.BlockSpec((PROJ_B, TBL), lambda b: (b, 0)),
      ],
      out_shape=[
          jax.ShapeDtypeStruct((N, HID), jnp.float32),
          jax.ShapeDtypeStruct((N, HID), jnp.float32),
          jax.ShapeDtypeStruct((N, TBL), jnp.float32),
      ],
  )(feat, w_q, w_k, w_v, w_pos1)


POOL_T = 6


def _flat_topk(d, qn):
  iota = lax.broadcasted_iota(jnp.int32, (qn, N), 1)
  kiota = lax.broadcasted_iota(jnp.int32, (qn, K_NN), 1)
  idxbuf = jnp.zeros((qn, K_NN), jnp.int32)

  def body(r, carry):
    d, idxbuf = carry
    m = jnp.min(d, axis=1, keepdims=True)
    j = jnp.min(jnp.where(d <= m, iota, N), axis=1, keepdims=True)
    idxbuf = jnp.where(kiota == r, j, idxbuf)
    d = jnp.where(iota == j, jnp.inf, d)
    return d, idxbuf

  _, idxbuf = lax.fori_loop(0, K_NN, body, (d, idxbuf))
  return idxbuf


def _topk_body(xq_ref, xyzt_ref, idx_ref):
  qn = TOPK_B
  ns = N // 128
  ng = 128
  xq = xq_ref[...]
  xt = xyzt_ref[...]
  sqc = jnp.sum(xt * xt, axis=0, keepdims=True)
  d = sqc - 2.0 * jnp.dot(xq, xt, preferred_element_type=jnp.float32)
  d3 = d.reshape(qn, ns, ng)
  siota3 = lax.broadcasted_iota(jnp.int32, (qn, ns, ng), 1).astype(jnp.float32)
  kiota = lax.broadcasted_iota(jnp.int32, (qn, K_NN), 1)
  giota = lax.broadcasted_iota(jnp.int32, (qn, ng), 1).astype(jnp.float32)

  masked = d3
  ms, jids = [], []
  for _ in range(POOL_T):
    mt = jnp.min(masked, axis=1)
    hit = masked == mt[:, None, :]
    at = jnp.min(jnp.where(hit, siota3, float(ns)), axis=1)
    masked = jnp.where(hit & (siota3 == at[:, None, :]), jnp.inf, masked)
    ms.append(mt)
    jids.append(at * ng + giota)
  pool = jnp.concatenate(ms, axis=1)
  jpool = jnp.concatenate(jids, axis=1)
  pw = ng * POOL_T
  big = jnp.float32(N)
  idxbuf = jnp.zeros((qn, K_NN), jnp.float32)

  def body(r, carry):
    pool, idxbuf = carry
    m = jnp.min(pool, axis=1, keepdims=True)
    hit = pool <= m
    j = jnp.min(jnp.where(hit, jpool, big), axis=1, keepdims=True)
    idxbuf = jnp.where(kiota == r, j, idxbuf)
    pool = jnp.where(hit & (jpool == j), jnp.inf, pool)
    return pool, idxbuf

  pool, idxbuf = lax.fori_loop(0, K_NN, body, (pool, idxbuf))
  idx_ref[...] = idxbuf.astype(jnp.int32)

  @pl.when(jnp.any(pool[:, (POOL_T - 1) * ng:] == jnp.inf))
  def _():
    idx_ref[...] = _flat_topk(d, qn)


def _topk(xyz, xyzt, nq):
  grid = (nq // TOPK_B,)
  return pl.pallas_call(
      _topk_body,
      grid=grid,
      in_specs=[
          pl.BlockSpec((TOPK_B, 3), lambda b: (b, 0)),
          pl.BlockSpec((3, N), lambda b: (0, 0)),
      ],
      out_specs=pl.BlockSpec((TOPK_B, K_NN), lambda b: (b, 0)),
      out_shape=jax.ShapeDtypeStruct((nq, K_NN), jnp.int32),
  )(xyz, xyzt)


def _gather(tbl, flat_idx):
  b_total = flat_idx.shape[0]
  b_per_w = b_total // NUM_WORKERS
  n_chunks = b_per_w // GATHER_CHUNK
  mesh = plsc.VectorSubcoreMesh(core_axis_name="c", subcore_axis_name="s")

  @functools.partial(
      pl.kernel,
      out_type=jax.ShapeDtypeStruct((b_total, TBL), jnp.float32),
      mesh=mesh,
      scratch_types=[
          pltpu.VMEM((GATHER_CHUNK,), jnp.int32),
          pltpu.VMEM((GATHER_CHUNK, TBL), jnp.float32),
          pltpu.SemaphoreType.DMA,
      ],
  )
  def gk(tbl_hbm, idx_hbm, out_hbm, idx_v, rows_v, sem):
    wid = lax.axis_index("s") * 2 + lax.axis_index("c")
    base = wid * b_per_w

    def body(i, carry):
      off = base + i * GATHER_CHUNK
      pltpu.sync_copy(idx_hbm.at[pl.ds(off, GATHER_CHUNK)], idx_v)
      pltpu.async_copy(tbl_hbm.at[idx_v], rows_v, sem).wait()
      pltpu.sync_copy(rows_v, out_hbm.at[pl.ds(off, GATHER_CHUNK)])
      return carry

    lax.fori_loop(0, n_chunks, body, 0)

  return gk(tbl, flat_idx)


def _mlp_body(g_ref, q_ref, pq_ref, bp1_ref, wp2_ref, bp2_ref,
              wa1_ref, ba1_ref, wa2_ref, ba2_ref, wc_ref, bc_ref, sem_ref):
  g = g_ref[...]
  nb = MLP_B
  kf = g[:, :, 0:HID].reshape(K_NN * nb, HID)
  vf = g[:, :, HID:2 * HID].reshape(K_NN * nb, HID)
  pj = g[:, :, 2 * HID:3 * HID]
  pq = pq_ref[...]
  h = pj - pq[None, :, :] + bp1_ref[...].reshape(1, 1, HID)
  ph = jnp.maximum(h, 0.0).reshape(K_NN * nb, HID)
  pos = (jnp.dot(ph, wp2_ref[...], preferred_element_type=jnp.float32)
         + bp2_ref[...])
  q = q_ref[...]
  qrep = jnp.broadcast_to(q[None, :, :], (K_NN, nb, HID)).reshape(
      K_NN * nb, HID)
  a0 = qrep - kf + pos
  ah = jnp.maximum(
      jnp.dot(a0, wa1_ref[...], preferred_element_type=jnp.float32)
      + ba1_ref[...], 0.0)
  al = (jnp.dot(ah, wa2_ref[...], preferred_element_type=jnp.float32)
        + ba2_ref[...]).reshape(K_NN, nb, HID)
  m = jnp.max(al, axis=0, keepdims=True)
  e = jnp.exp(al - m)
  s = jnp.sum(e, axis=0, keepdims=True)
  attn = e / s
  vp = vf + pos
  out = jnp.sum(attn * vp.reshape(K_NN, nb, HID), axis=0)
  sem_ref[...] = (jnp.dot(out, wc_ref[...], preferred_element_type=jnp.float32)
                  + bc_ref[...])


def _mlp(g3, q, pq, bp1, wp2, bp2, wa1, ba1, wa2, ba2, wc, bc):
  nq = q.shape[0]
  grid = (nq // MLP_B,)
  full = lambda r, c: pl.BlockSpec((r, c), lambda b: (0, 0))
  return pl.pallas_call(
      _mlp_body,
      grid=grid,
      in_specs=[
          pl.BlockSpec((K_NN, MLP_B, TBL), lambda b: (0, b, 0)),
          pl.BlockSpec((MLP_B, HID), lambda b: (b, 0)),
          pl.BlockSpec((MLP_B, HID), lambda b: (b, 0)),
          full(1, HID), full(HID, HID), full(1, HID),
          full(HID, HID), full(1, HID), full(HID, HID), full(1, HID),
          full(HID, N_CLS), full(1, N_CLS),
      ],
      out_specs=pl.BlockSpec((MLP_B, N_CLS), lambda b: (b, 0)),
      out_shape=jax.ShapeDtypeStruct((nq, N_CLS), jnp.float32),
  )(g3, q, pq, bp1, wp2, bp2, wa1, ba1, wa2, ba2, wc, bc)


def kernel(feat, w_q, w_k, w_v, w_pos1, b_pos1, w_pos2, b_pos2,
           w_attn1, b_attn1, w_attn2, b_attn2, w_cls, b_cls):
  xyz = feat[:, :3]
  xyzt = xyz.T
  q, pq, tbl = _proj(feat, w_q, w_k, w_v, w_pos1)
  r1 = lambda b: b.reshape(1, -1)
  nh = N // 4
  outs = []
  for h in range(4):
    rows = slice(h * nh, (h + 1) * nh)
    idx = _topk(xyz[rows], xyzt, nh)
    flat_idx = idx.T.reshape(-1)
    gath = _gather(tbl, flat_idx)
    g3 = gath.reshape(K_NN, nh, TBL)
    outs.append(_mlp(g3, q[rows], pq[rows], r1(b_pos1), w_pos2, r1(b_pos2),
                     w_attn1, r1(b_attn1), w_attn2, r1(b_attn2),
                     w_cls, r1(b_cls)))
  return jnp.concatenate(outs, axis=0)


# speedup vs baseline: 1.1285x; 1.0000x over previous
import functools

import jax
import jax.numpy as jnp
from jax import lax
from jax.experimental import pallas as pl
from jax.experimental.pallas import tpu as pltpu
from jax.experimental.pallas import tpu_sc as plsc

N = 8192
HID = 64
K_NN = 36
N_CLS = 17
TBL = 256

PROJ_B = 512
TOPK_B = 256
MLP_B = 128

NUM_WORKERS = 32
GATHER_CHUNK = 128


def _proj_body(feat_ref, wq_ref, wk_ref, wv_ref, wp1_ref, q_ref, pq_ref,
               tbl_ref):
  f = feat_ref[...]
  q_ref[...] = jnp.dot(f, wq_ref[...], preferred_element_type=jnp.float32)
  kf = jnp.dot(f, wk_ref[...], preferred_element_type=jnp.float32)
  vf = jnp.dot(f, wv_ref[...], preferred_element_type=jnp.float32)
  p = jnp.dot(f[:, 0:3], wp1_ref[...], preferred_element_type=jnp.float32)
  pq_ref[...] = p
  tbl_ref[:, 0:HID] = kf
  tbl_ref[:, HID:2 * HID] = vf
  tbl_ref[:, 2 * HID:3 * HID] = p
  tbl_ref[:, 3 * HID:TBL] = jnp.zeros((f.shape[0], HID), jnp.float32)


def _proj(feat, w_q, w_k, w_v, w_pos1):
  grid = (N // PROJ_B,)
  return pl.pallas_call(
      _proj_body,
      grid=grid,
      in_specs=[
          pl.BlockSpec((PROJ_B, 6), lambda b: (b, 0)),
          pl.BlockSpec((6, HID), lambda b: (0, 0)),
          pl.BlockSpec((6, HID), lambda b: (0, 0)),
          pl.BlockSpec((6, HID), lambda b: (0, 0)),
          pl.BlockSpec((3, HID), lambda b: (0, 0)),
      ],
      out_specs=[
          pl.BlockSpec((PROJ_B, HID), lambda b: (b, 0)),
          pl.BlockSpec((PROJ_B, HID), lambda b: (b, 0)),
          pl.BlockSpec((PROJ_B, TBL), lambda b: (b, 0)),
      ],
      out_shape=[
          jax.ShapeDtypeStruct((N, HID), jnp.float32),
          jax.ShapeDtypeStruct((N, HID), jnp.float32),
          jax.ShapeDtypeStruct((N, TBL), jnp.float32),
      ],
  )(feat, w_q, w_k, w_v, w_pos1)


POOL_T = 6


def _flat_topk(d, qn):
  iota = lax.broadcasted_iota(jnp.int32, (qn, N), 1)
  kiota = lax.broadcasted_iota(jnp.int32, (qn, K_NN), 1)
  idxbuf = jnp.zeros((qn, K_NN), jnp.int32)

  def body(r, carry):
    d, idxbuf = carry
    m = jnp.min(d, axis=1, keepdims=True)
    j = jnp.min(jnp.where(d <= m, iota, N), axis=1, keepdims=True)
    idxbuf = jnp.where(kiota == r, j, idxbuf)
    d = jnp.where(iota == j, jnp.inf, d)
    return d, idxbuf

  _, idxbuf = lax.fori_loop(0, K_NN, body, (d, idxbuf))
  return idxbuf


def _topk_body(xq_ref, xyzt_ref, idx_ref):
  qn = TOPK_B
  ns = N // 128
  ng = 128
  xq = xq_ref[...]
  xt = xyzt_ref[...]
  sqc = jnp.sum(xt * xt, axis=0, keepdims=True)
  d = sqc - 2.0 * jnp.dot(xq, xt, preferred_element_type=jnp.float32)
  d3 = d.reshape(qn, ns, ng)
  siota3 = lax.broadcasted_iota(jnp.int32, (qn, ns, ng), 1).astype(jnp.float32)
  kiota = lax.broadcasted_iota(jnp.int32, (qn, K_NN), 1)
  giota = lax.broadcasted_iota(jnp.int32, (qn, ng), 1).astype(jnp.float32)

  masked = d3
  ms, jids = [], []
  for _ in range(POOL_T):
    mt = jnp.min(masked, axis=1)
    hit = masked == mt[:, None, :]
    at = jnp.min(jnp.where(hit, siota3, float(ns)), axis=1)
    masked = jnp.where(hit & (siota3 == at[:, None, :]), jnp.inf, masked)
    ms.append(mt)
    jids.append(at * ng + giota)
  pool = jnp.concatenate(ms, axis=1)
  jpool = jnp.concatenate(jids, axis=1)
  pw = ng * POOL_T
  big = jnp.float32(N)
  idxbuf = jnp.zeros((qn, K_NN), jnp.float32)

  def body(r, carry):
    pool, idxbuf = carry
    m = jnp.min(pool, axis=1, keepdims=True)
    hit = pool <= m
    j = jnp.min(jnp.where(hit, jpool, big), axis=1, keepdims=True)
    idxbuf = jnp.where(kiota == r, j, idxbuf)
    pool = jnp.where(hit & (jpool == j), jnp.inf, pool)
    return pool, idxbuf

  pool, idxbuf = lax.fori_loop(0, K_NN, body, (pool, idxbuf))
  idx_ref[...] = idxbuf.astype(jnp.int32)

  @pl.when(jnp.any(pool[:, (POOL_T - 1) * ng:] == jnp.inf))
  def _():
    idx_ref[...] = _flat_topk(d, qn)


def _topk(xyz, xyzt, nq):
  grid = (nq // TOPK_B,)
  return pl.pallas_call(
      _topk_body,
      grid=grid,
      in_specs=[
          pl.BlockSpec((TOPK_B, 3), lambda b: (b, 0)),
          pl.BlockSpec((3, N), lambda b: (0, 0)),
      ],
      out_specs=pl.BlockSpec((TOPK_B, K_NN), lambda b: (b, 0)),
      out_shape=jax.ShapeDtypeStruct((nq, K_NN), jnp.int32),
  )(xyz, xyzt)


def _gather(tbl, flat_idx):
  b_total = flat_idx.shape[0]
  b_per_w = b_total // NUM_WORKERS
  n_chunks = b_per_w // GATHER_CHUNK
  mesh = plsc.VectorSubcoreMesh(core_axis_name="c", subcore_axis_name="s")

  @functools.partial(
      pl.kernel,
      out_type=jax.ShapeDtypeStruct((b_total, TBL), jnp.float32),
      mesh=mesh,
      scratch_types=[
          pltpu.VMEM((GATHER_CHUNK,), jnp.int32),
          pltpu.VMEM((GATHER_CHUNK, TBL), jnp.float32),
          pltpu.SemaphoreType.DMA,
      ],
  )
  def gk(tbl_hbm, idx_hbm, out_hbm, idx_v, rows_v, sem):
    wid = lax.axis_index("s") * 2 + lax.axis_index("c")
    base = wid * b_per_w

    def body(i, carry):
      off = base + i * GATHER_CHUNK
      pltpu.sync_copy(idx_hbm.at[pl.ds(off, GATHER_CHUNK)], idx_v)
      pltpu.async_copy(tbl_hbm.at[idx_v], rows_v, sem).wait()
      pltpu.sync_copy(rows_v, out_hbm.at[pl.ds(off, GATHER_CHUNK)])
      return carry

    lax.fori_loop(0, n_chunks, body, 0)

  return gk(tbl, flat_idx)


def _mlp_body(g_ref, q_ref, pq_ref, bp1_ref, wp2_ref, bp2_ref,
              wa1_ref, ba1_ref, wa2_ref, ba2_ref, wc_ref, bc_ref, sem_ref):
  g = g_ref[...]
  nb = MLP_B
  kf = g[:, :, 0:HID].reshape(K_NN * nb, HID)
  vf = g[:, :, HID:2 * HID].reshape(K_NN * nb, HID)
  pj = g[:, :, 2 * HID:3 * HID]
  pq = pq_ref[...]
  h = pj - pq[None, :, :] + bp1_ref[...].reshape(1, 1, HID)
  ph = jnp.maximum(h, 0.0).reshape(K_NN * nb, HID)
  pos = (jnp.dot(ph, wp2_ref[...], preferred_element_type=jnp.float32)
         + bp2_ref[...])
  q = q_ref[...]
  qrep = jnp.broadcast_to(q[None, :, :], (K_NN, nb, HID)).reshape(
      K_NN * nb, HID)
  a0 = qrep - kf + pos
  ah = jnp.maximum(
      jnp.dot(a0, wa1_ref[...], preferred_element_type=jnp.float32)
      + ba1_ref[...], 0.0)
  al = (jnp.dot(ah, wa2_ref[...], preferred_element_type=jnp.float32)
        + ba2_ref[...]).reshape(K_NN, nb, HID)
  m = jnp.max(al, axis=0, keepdims=True)
  e = jnp.exp(al - m)
  s = jnp.sum(e, axis=0, keepdims=True)
  attn = e / s
  vp = vf + pos
  out = jnp.sum(attn * vp.reshape(K_NN, nb, HID), axis=0)
  sem_ref[...] = (jnp.dot(out, wc_ref[...], preferred_element_type=jnp.float32)
                  + bc_ref[...])


def _mlp(g3, q, pq, bp1, wp2, bp2, wa1, ba1, wa2, ba2, wc, bc):
  nq = q.shape[0]
  grid = (nq // MLP_B,)
  full = lambda r, c: pl.BlockSpec((r, c), lambda b: (0, 0))
  return pl.pallas_call(
      _mlp_body,
      grid=grid,
      in_specs=[
          pl.BlockSpec((K_NN, MLP_B, TBL), lambda b: (0, b, 0)),
          pl.BlockSpec((MLP_B, HID), lambda b: (b, 0)),
          pl.BlockSpec((MLP_B, HID), lambda b: (b, 0)),
          full(1, HID), full(HID, HID), full(1, HID),
          full(HID, HID), full(1, HID), full(HID, HID), full(1, HID),
          full(HID, N_CLS), full(1, N_CLS),
      ],
      out_specs=pl.BlockSpec((MLP_B, N_CLS), lambda b: (b, 0)),
      out_shape=jax.ShapeDtypeStruct((nq, N_CLS), jnp.float32),
  )(g3, q, pq, bp1, wp2, bp2, wa1, ba1, wa2, ba2, wc, bc)


def kernel(feat, w_q, w_k, w_v, w_pos1, b_pos1, w_pos2, b_pos2,
           w_attn1, b_attn1, w_attn2, b_attn2, w_cls, b_cls):
  xyz = feat[:, :3]
  xyzt = xyz.T
  q, pq, tbl = _proj(feat, w_q, w_k, w_v, w_pos1)
  r1 = lambda b: b.reshape(1, -1)
  nh = N // 4
  slices = [slice(h * nh, (h + 1) * nh) for h in range(4)]
  idxs = [_topk(xyz[rows], xyzt, nh) for rows in slices]
  gaths = [_gather(tbl, idx.T.reshape(-1)) for idx in idxs]
  outs = [
      _mlp(g.reshape(K_NN, nh, TBL), q[rows], pq[rows], r1(b_pos1), w_pos2,
           r1(b_pos2), w_attn1, r1(b_attn1), w_attn2, r1(b_attn2),
           w_cls, r1(b_cls))
      for g, rows in zip(gaths, slices)
  ]
  return jnp.concatenate(outs, axis=0)


# speedup vs baseline: 1.1925x; 1.0567x over previous
import functools

import jax
import jax.numpy as jnp
from jax import lax
from jax.experimental import pallas as pl
from jax.experimental.pallas import tpu as pltpu
from jax.experimental.pallas import tpu_sc as plsc

N = 8192
HID = 64
K_NN = 36
N_CLS = 17
TBL = 256

PROJ_B = 512
TOPK_B = 256
MLP_B = 128

NUM_WORKERS = 32
GATHER_CHUNK = 128


def _proj_body(feat_ref, wq_ref, wk_ref, wv_ref, wp1_ref, q_ref, pq_ref,
               tbl_ref):
  f = feat_ref[...]
  q_ref[...] = jnp.dot(f, wq_ref[...], preferred_element_type=jnp.float32)
  kf = jnp.dot(f, wk_ref[...], preferred_element_type=jnp.float32)
  vf = jnp.dot(f, wv_ref[...], preferred_element_type=jnp.float32)
  p = jnp.dot(f[:, 0:3], wp1_ref[...], preferred_element_type=jnp.float32)
  pq_ref[...] = p
  tbl_ref[:, 0:HID] = kf
  tbl_ref[:, HID:2 * HID] = vf
  tbl_ref[:, 2 * HID:3 * HID] = p
  tbl_ref[:, 3 * HID:TBL] = jnp.zeros((f.shape[0], HID), jnp.float32)


def _proj(feat, w_q, w_k, w_v, w_pos1):
  grid = (N // PROJ_B,)
  return pl.pallas_call(
      _proj_body,
      grid=grid,
      in_specs=[
          pl.BlockSpec((PROJ_B, 6), lambda b: (b, 0)),
          pl.BlockSpec((6, HID), lambda b: (0, 0)),
          pl.BlockSpec((6, HID), lambda b: (0, 0)),
          pl.BlockSpec((6, HID), lambda b: (0, 0)),
          pl.BlockSpec((3, HID), lambda b: (0, 0)),
      ],
      out_specs=[
          pl.BlockSpec((PROJ_B, HID), lambda b: (b, 0)),
          pl.BlockSpec((PROJ_B, HID), lambda b: (b, 0)),
          pl.BlockSpec((PROJ_B, TBL), lambda b: (b, 0)),
      ],
      out_shape=[
          jax.ShapeDtypeStruct((N, HID), jnp.float32),
          jax.ShapeDtypeStruct((N, HID), jnp.float32),
          jax.ShapeDtypeStruct((N, TBL), jnp.float32),
      ],
  )(feat, w_q, w_k, w_v, w_pos1)


POOL_T = 6


def _flat_topk(d, qn):
  iota = lax.broadcasted_iota(jnp.int32, (qn, N), 1)
  kiota = lax.broadcasted_iota(jnp.int32, (qn, K_NN), 1)
  idxbuf = jnp.zeros((qn, K_NN), jnp.int32)

  def body(r, carry):
    d, idxbuf = carry
    m = jnp.min(d, axis=1, keepdims=True)
    j = jnp.min(jnp.where(d <= m, iota, N), axis=1, keepdims=True)
    idxbuf = jnp.where(kiota == r, j, idxbuf)
    d = jnp.where(iota == j, jnp.inf, d)
    return d, idxbuf

  _, idxbuf = lax.fori_loop(0, K_NN, body, (d, idxbuf))
  return idxbuf


def _topk_body(xq_ref, xyzt_ref, idx_ref):
  qn = TOPK_B
  ns = N // 128
  ng = 128
  xq = xq_ref[...]
  xt = xyzt_ref[...]
  sqc = jnp.sum(xt * xt, axis=0, keepdims=True)
  d = sqc - 2.0 * jnp.dot(xq, xt, preferred_element_type=jnp.float32)
  d3 = d.reshape(qn, ns, ng)
  siota3 = lax.broadcasted_iota(jnp.int32, (qn, ns, ng), 1).astype(jnp.float32)
  kiota = lax.broadcasted_iota(jnp.int32, (qn, K_NN), 1)
  giota = lax.broadcasted_iota(jnp.int32, (qn, ng), 1).astype(jnp.float32)

  masked = d3
  ms, jids = [], []
  for _ in range(POOL_T):
    mt = jnp.min(masked, axis=1)
    hit = masked == mt[:, None, :]
    at = jnp.min(jnp.where(hit, siota3, float(ns)), axis=1)
    masked = jnp.where(hit & (siota3 == at[:, None, :]), jnp.inf, masked)
    ms.append(mt)
    jids.append(at * ng + giota)
  pool = jnp.concatenate(ms, axis=1)
  jpool = jnp.concatenate(jids, axis=1)
  pw = ng * POOL_T
  big = jnp.float32(N)
  idxbuf = jnp.zeros((qn, K_NN), jnp.float32)

  def body(r, carry):
    pool, idxbuf = carry
    m = jnp.min(pool, axis=1, keepdims=True)
    j = jnp.min(jnp.where(pool <= m, jpool, big), axis=1, keepdims=True)
    idxbuf = jnp.where(kiota == r, j, idxbuf)
    pool = jnp.where(jpool == j, jnp.inf, pool)
    return pool, idxbuf

  pool, idxbuf = lax.fori_loop(0, K_NN, body, (pool, idxbuf))
  idx_ref[...] = idxbuf.astype(jnp.int32)

  @pl.when(jnp.any(pool[:, (POOL_T - 1) * ng:] == jnp.inf))
  def _():
    idx_ref[...] = _flat_topk(d, qn)


def _topk(xyz, xyzt, nq):
  grid = (nq // TOPK_B,)
  return pl.pallas_call(
      _topk_body,
      grid=grid,
      in_specs=[
          pl.BlockSpec((TOPK_B, 3), lambda b: (b, 0)),
          pl.BlockSpec((3, N), lambda b: (0, 0)),
      ],
      out_specs=pl.BlockSpec((TOPK_B, K_NN), lambda b: (b, 0)),
      out_shape=jax.ShapeDtypeStruct((nq, K_NN), jnp.int32),
  )(xyz, xyzt)


def _gather(tbl, flat_idx):
  b_total = flat_idx.shape[0]
  b_per_w = b_total // NUM_WORKERS
  n_chunks = b_per_w // GATHER_CHUNK
  mesh = plsc.VectorSubcoreMesh(core_axis_name="c", subcore_axis_name="s")

  @functools.partial(
      pl.kernel,
      out_type=jax.ShapeDtypeStruct((b_total, TBL), jnp.float32),
      mesh=mesh,
      scratch_types=[
          pltpu.VMEM((GATHER_CHUNK,), jnp.int32),
          pltpu.VMEM((GATHER_CHUNK, TBL), jnp.float32),
          pltpu.SemaphoreType.DMA,
      ],
  )
  def gk(tbl_hbm, idx_hbm, out_hbm, idx_v, rows_v, sem):
    wid = lax.axis_index("s") * 2 + lax.axis_index("c")
    base = wid * b_per_w

    def body(i, carry):
      off = base + i * GATHER_CHUNK
      pltpu.sync_copy(idx_hbm.at[pl.ds(off, GATHER_CHUNK)], idx_v)
      pltpu.async_copy(tbl_hbm.at[idx_v], rows_v, sem).wait()
      pltpu.sync_copy(rows_v, out_hbm.at[pl.ds(off, GATHER_CHUNK)])
      return carry

    lax.fori_loop(0, n_chunks, body, 0)

  return gk(tbl, flat_idx)


def _mlp_body(g_ref, q_ref, pq_ref, bp1_ref, wp2_ref, bp2_ref,
              wa1_ref, ba1_ref, wa2_ref, ba2_ref, wc_ref, bc_ref, sem_ref):
  g = g_ref[...]
  nb = MLP_B
  kf = g[:, :, 0:HID].reshape(K_NN * nb, HID)
  vf = g[:, :, HID:2 * HID].reshape(K_NN * nb, HID)
  pj = g[:, :, 2 * HID:3 * HID]
  pq = pq_ref[...]
  h = pj - pq[None, :, :] + bp1_ref[...].reshape(1, 1, HID)
  ph = jnp.maximum(h, 0.0).reshape(K_NN * nb, HID)
  pos = (jnp.dot(ph, wp2_ref[...], preferred_element_type=jnp.float32)
         + bp2_ref[...])
  q = q_ref[...]
  qrep = jnp.broadcast_to(q[None, :, :], (K_NN, nb, HID)).reshape(
      K_NN * nb, HID)
  a0 = qrep - kf + pos
  ah = jnp.maximum(
      jnp.dot(a0, wa1_ref[...], preferred_element_type=jnp.float32)
      + ba1_ref[...], 0.0)
  al = (jnp.dot(ah, wa2_ref[...], preferred_element_type=jnp.float32)
        + ba2_ref[...]).reshape(K_NN, nb, HID)
  m = jnp.max(al, axis=0, keepdims=True)
  e = jnp.exp(al - m)
  s = jnp.sum(e, axis=0, keepdims=True)
  attn = e / s
  vp = vf + pos
  out = jnp.sum(attn * vp.reshape(K_NN, nb, HID), axis=0)
  sem_ref[...] = (jnp.dot(out, wc_ref[...], preferred_element_type=jnp.float32)
                  + bc_ref[...])


def _mlp(g3, q, pq, bp1, wp2, bp2, wa1, ba1, wa2, ba2, wc, bc):
  nq = q.shape[0]
  grid = (nq // MLP_B,)
  full = lambda r, c: pl.BlockSpec((r, c), lambda b: (0, 0))
  return pl.pallas_call(
      _mlp_body,
      grid=grid,
      in_specs=[
          pl.BlockSpec((K_NN, MLP_B, TBL), lambda b: (0, b, 0)),
          pl.BlockSpec((MLP_B, HID), lambda b: (b, 0)),
          pl.BlockSpec((MLP_B, HID), lambda b: (b, 0)),
          full(1, HID), full(HID, HID), full(1, HID),
          full(HID, HID), full(1, HID), full(HID, HID), full(1, HID),
          full(HID, N_CLS), full(1, N_CLS),
      ],
      out_specs=pl.BlockSpec((MLP_B, N_CLS), lambda b: (b, 0)),
      out_shape=jax.ShapeDtypeStruct((nq, N_CLS), jnp.float32),
  )(g3, q, pq, bp1, wp2, bp2, wa1, ba1, wa2, ba2, wc, bc)


def kernel(feat, w_q, w_k, w_v, w_pos1, b_pos1, w_pos2, b_pos2,
           w_attn1, b_attn1, w_attn2, b_attn2, w_cls, b_cls):
  xyz = feat[:, :3]
  xyzt = xyz.T
  q, pq, tbl = _proj(feat, w_q, w_k, w_v, w_pos1)
  r1 = lambda b: b.reshape(1, -1)
  nh = N // 4
  slices = [slice(h * nh, (h + 1) * nh) for h in range(4)]
  idxs = [_topk(xyz[rows], xyzt, nh) for rows in slices]
  gaths = [_gather(tbl, idx.T.reshape(-1)) for idx in idxs]
  outs = [
      _mlp(g.reshape(K_NN, nh, TBL), q[rows], pq[rows], r1(b_pos1), w_pos2,
           r1(b_pos2), w_attn1, r1(b_attn1), w_attn2, r1(b_attn2),
           w_cls, r1(b_cls))
      for g, rows in zip(gaths, slices)
  ]
  return jnp.concatenate(outs, axis=0)
